# baseline (speedup 1.0000x reference)
import functools

import jax
import jax.numpy as jnp
import numpy as np
from jax import lax
from jax.experimental import pallas as pl
from jax.experimental.pallas import tpu as pltpu
from jax.experimental.pallas import tpu_sc as plsc

N = 10000
E = 320000
F = 128
G = 64
N_OUT = 10

NC = 2
NS = 16
NW = NC * NS

IDXW = 128
PADROWS = -(-E // (NW * IDXW * 8)) * NW * 8
EPAD = PADROWS * IDXW
RPW = PADROWS // NW
PADN = 10240
RPT = PADN // NS
CH = 40


def _sc_body(h_hbm, src_hbm, dst_hbm, z_hbm, out_hbm, acc, idx_s, idx_d,
             rows0, rows1, rows2, rows3,
             semg0, semg1, semg2, semg3, sems0, sems1, sems2, sems3):
    cid = lax.axis_index("c")
    sid = lax.axis_index("s")
    wid = sid * NC + cid

    with jax.named_scope("ph_zero"):
        pltpu.sync_copy(z_hbm, acc.at[pl.ds(sid * RPT, RPT)])

    base = wid * RPW
    plsc.subcore_barrier()

    bufs = (rows0, rows1, rows2, rows3)
    gsems = (semg0, semg1, semg2, semg3)
    ssems = (sems0, sems1, sems2, sems3)
    NT = 2 * CH

    def chunk(ci, carry):
        hbase = base + ci * CH
        pltpu.sync_copy(src_hbm.at[pl.ds(hbase, CH)], idx_s)
        pltpu.sync_copy(dst_hbm.at[pl.ds(hbase, CH)], idx_d)

        def sidx(jj, off):
            return idx_s.at[jj, pl.ds(off, 64)]

        def didx(jj, off):
            return idx_d.at[jj, pl.ds(off, 64)]

        pltpu.async_copy(h_hbm.at[sidx(0, 0)], rows0, semg0)
        pltpu.async_copy(h_hbm.at[sidx(0, 64)], rows1, semg1)

        def step(i, c):
            for k in range(4):
                tt = 4 * i + k
                jj = 2 * i + k // 2
                off = (k % 2) * 64
                kn = (k + 2) % 4
                pltpu.make_async_copy(h_hbm.at[sidx(jj, off)], bufs[k], gsems[k]).wait()
                pltpu.async_copy(bufs[k], acc.at[didx(jj, off)], ssems[k], add=True)

                @pl.when(tt + 2 < NT)
                def _():
                    @pl.when(tt >= 2)
                    def _():
                        pltpu.make_async_copy(
                            bufs[kn], acc.at[didx(jj, off)], ssems[kn]).wait()

                    pltpu.async_copy(h_hbm.at[sidx(jj + 1, off)], bufs[kn], gsems[kn])
            return c

        lax.fori_loop(0, NT // 4, step, carry)
        for k in range(4):
            pltpu.make_async_copy(bufs[k], acc.at[didx(0, 0)], ssems[k]).wait()
        return carry

    with jax.named_scope("ph_edges"):
        lax.fori_loop(0, RPW // CH, chunk, 0)
    with jax.named_scope("ph_bar"):
        plsc.subcore_barrier()

    with jax.named_scope("ph_writeout"):
        pltpu.sync_copy(acc.at[pl.ds(sid * RPT, RPT)],
                        out_hbm.at[pl.ds(cid * PADN + sid * RPT, RPT)])


@jax.jit
def _sc_segment_sum(h, src2d, dst2d, zrows):
    mesh = plsc.VectorSubcoreMesh(core_axis_name="c", subcore_axis_name="s")
    k = pl.kernel(
        _sc_body,
        out_type=jax.ShapeDtypeStruct((NC * PADN, F), jnp.float32),
        mesh=mesh,
        scratch_types=[
            pltpu.VMEM_SHARED((PADN, F), jnp.float32),
            pltpu.VMEM((CH, IDXW), jnp.int32),
            pltpu.VMEM((CH, IDXW), jnp.int32),
        ] + [pltpu.VMEM((64, F), jnp.float32)] * 4
          + [pltpu.SemaphoreType.DMA] * 8,
    )
    return k(h, src2d, dst2d, zrows)


def _layer_body(h_ref, agg_ref, w1, b1, w2, b2, w3, b3, o_ref):
    z = h_ref[...] + agg_ref[0] + agg_ref[1]
    z = jnp.maximum(jnp.dot(z, w1[...], preferred_element_type=jnp.float32) + b1[...], 0.0)
    z = jnp.maximum(jnp.dot(z, w2[...], preferred_element_type=jnp.float32) + b2[...], 0.0)
    o_ref[...] = jnp.dot(z, w3[...], preferred_element_type=jnp.float32) + b3[...]


BLK = 2000
NBLK = N // BLK


def _tc_layer(h, agg2, w1, b1, w2, b2, w3, b3):
    wspec = pl.BlockSpec((F, F), lambda i: (0, 0))
    bspec = pl.BlockSpec((1, F), lambda i: (0, 0))
    return pl.pallas_call(
        _layer_body,
        grid=(NBLK,),
        in_specs=[
            pl.BlockSpec((BLK, F), lambda i: (i, 0)),
            pl.BlockSpec((NC, BLK, F), lambda i: (0, i, 0)),
            wspec, bspec, wspec, bspec, wspec, bspec,
        ],
        out_specs=pl.BlockSpec((BLK, F), lambda i: (i, 0)),
        out_shape=jax.ShapeDtypeStruct((N, F), jnp.float32),
    )(h, agg2, w1, b1, w2, b2, w3, b3)


def _pool_body(h_ref, seg_ref, d1w, d1b, d2w, d2b, o_ref, sums, cnts):
    i = pl.program_id(0)

    @pl.when(i == 0)
    def _():
        sums[...] = jnp.zeros((G, F), jnp.float32)
        cnts[...] = jnp.zeros((G, F), jnp.float32)

    onehot = (seg_ref[...] == lax.broadcasted_iota(jnp.int32, (1, G), 1)
              ).astype(jnp.float32)
    cdims = (((0,), (0,)), ((), ()))
    sums[...] += lax.dot_general(onehot, h_ref[...], cdims,
                                 preferred_element_type=jnp.float32)
    cnts[...] += lax.dot_general(onehot, jnp.ones((BLK, F), jnp.float32), cdims,
                                 preferred_element_type=jnp.float32)

    @pl.when(i == NBLK - 1)
    def _():
        pooled = sums[...] / jnp.maximum(cnts[...], 1.0)
        o1 = jnp.maximum(
            jnp.dot(pooled, d1w[...], preferred_element_type=jnp.float32) + d1b[...], 0.0)
        logits = jnp.dot(o1, d2w[...], preferred_element_type=jnp.float32) + d2b[...]
        mask = lax.broadcasted_iota(jnp.int32, (G, F), 1) < N_OUT
        logits = jnp.where(mask, logits, -1e30)
        m = jnp.max(logits, axis=1, keepdims=True)
        e = jnp.exp(logits - m)
        o_ref[...] = e / jnp.sum(e, axis=1, keepdims=True)


def _tc_pool_head(h, segf, d1w, d1b, d2wp, d2bp):
    wspec = pl.BlockSpec((F, F), lambda i: (0, 0))
    bspec = pl.BlockSpec((1, F), lambda i: (0, 0))
    return pl.pallas_call(
        _pool_body,
        grid=(NBLK,),
        in_specs=[
            pl.BlockSpec((BLK, F), lambda i: (i, 0)),
            pl.BlockSpec((BLK, 1), lambda i: (i, 0)),
            wspec, bspec, wspec, bspec,
        ],
        out_specs=pl.BlockSpec((G, F), lambda i: (0, 0)),
        out_shape=jax.ShapeDtypeStruct((G, F), jnp.float32),
        scratch_shapes=[
            pltpu.VMEM((G, F), jnp.float32),
            pltpu.VMEM((G, F), jnp.float32),
        ],
    )(h, segf, d1w, d1b, d2wp, d2bp)


def kernel(x, convW1, convb1, convW2, convb2, convW3, convb3,
           d1W, d1b, d2W, d2b, edge_index, seg):
    src = edge_index[0]
    dst = edge_index[1]
    ar = np.arange(EPAD - E, dtype=np.int32)
    pad_src = jnp.asarray((ar % N).reshape(-1, IDXW))
    pad_dst = jnp.asarray((N + ar % (PADN - N)).reshape(-1, IDXW))
    srcp = jnp.concatenate([src.reshape(E // IDXW, IDXW), pad_src])
    dstp = jnp.concatenate([dst.reshape(E // IDXW, IDXW), pad_dst])
    zrows = jnp.zeros((RPT, F), jnp.float32)

    h = x
    for l in range(3):
        aggp = _sc_segment_sum(h, srcp, dstp, zrows)
        agg2 = aggp.reshape(NC, PADN, F)
        h = _tc_layer(h, agg2,
                      convW1[l], convb1[l].reshape(1, F),
                      convW2[l], convb2[l].reshape(1, F),
                      convW3[l], convb3[l].reshape(1, F))

    segf = seg.reshape(N, 1)
    d2wp = jnp.pad(d2W, ((0, 0), (0, F - N_OUT)))
    d2bp = jnp.pad(d2b, (0, F - N_OUT)).reshape(1, F)
    out = _tc_pool_head(h, segf, d1W, d1b.reshape(1, F), d2wp, d2bp)
    return out[:, :N_OUT]


# speedup vs baseline: 1.0940x; 1.0940x over previous
import functools

import jax
import jax.numpy as jnp
import numpy as np
from jax import lax
from jax.experimental import pallas as pl
from jax.experimental.pallas import tpu as pltpu
from jax.experimental.pallas import tpu_sc as plsc

N = 10000
E = 320000
F = 128
G = 64
N_OUT = 10

NC = 2
NS = 16
NW = NC * NS

IDXW = 128
PADROWS = -(-E // (NW * IDXW * 8)) * NW * 8
EPAD = PADROWS * IDXW
RPW = PADROWS // NW
PADN = 10240
RPT = PADN // NS
CH = 40


def _sc_body(h_hbm, src_hbm, dst_hbm, z_hbm, out_hbm, acc, idx_s, idx_d,
             rows0, rows1, semg0, semg1):
    cid = lax.axis_index("c")
    sid = lax.axis_index("s")
    wid = sid * NC + cid

    with jax.named_scope("ph_zero"):
        pltpu.sync_copy(z_hbm, acc.at[pl.ds(sid * RPT, RPT)])

    base = wid * RPW
    plsc.subcore_barrier()

    def chunk(ci, carry):
        hbase = base + ci * CH
        pltpu.sync_copy(src_hbm.at[pl.ds(hbase, CH)], idx_s)
        pltpu.sync_copy(dst_hbm.at[pl.ds(hbase, CH)], idx_d)
        pltpu.async_copy(h_hbm.at[idx_s.at[0]], rows0, semg0)
        pltpu.async_copy(h_hbm.at[idx_s.at[1]], rows1, semg1)

        def step(i, c):
            j = 2 * i
            for off, rows, sem in ((0, rows0, semg0), (1, rows1, semg1)):
                pltpu.make_async_copy(h_hbm.at[idx_s.at[j + off]], rows, sem).wait()
                pltpu.sync_copy(rows, acc.at[idx_d.at[j + off]], add=True)

                @pl.when(j + off + 2 < CH)
                def _():
                    pltpu.async_copy(h_hbm.at[idx_s.at[j + off + 2]], rows, sem)
            return c

        return lax.fori_loop(0, CH // 2, step, carry)

    with jax.named_scope("ph_edges"):
        lax.fori_loop(0, RPW // CH, chunk, 0)
    with jax.named_scope("ph_bar"):
        plsc.subcore_barrier()

    with jax.named_scope("ph_writeout"):
        pltpu.sync_copy(acc.at[pl.ds(sid * RPT, RPT)],
                        out_hbm.at[pl.ds(cid * PADN + sid * RPT, RPT)])


@jax.jit
def _sc_segment_sum(h, src2d, dst2d, zrows):
    mesh = plsc.VectorSubcoreMesh(core_axis_name="c", subcore_axis_name="s")
    k = pl.kernel(
        _sc_body,
        out_type=jax.ShapeDtypeStruct((NC * PADN, F), jnp.float32),
        mesh=mesh,
        scratch_types=[
            pltpu.VMEM_SHARED((PADN, F), jnp.float32),
            pltpu.VMEM((CH, IDXW), jnp.int32),
            pltpu.VMEM((CH, IDXW), jnp.int32),
        ] + [pltpu.VMEM((IDXW, F), jnp.float32)] * 2
          + [pltpu.SemaphoreType.DMA] * 2,
    )
    return k(h, src2d, dst2d, zrows)


def _layer_body(h_ref, agg_ref, w1, b1, w2, b2, w3, b3, o_ref):
    z = h_ref[...] + agg_ref[0] + agg_ref[1]
    z = jnp.maximum(jnp.dot(z, w1[...], preferred_element_type=jnp.float32) + b1[...], 0.0)
    z = jnp.maximum(jnp.dot(z, w2[...], preferred_element_type=jnp.float32) + b2[...], 0.0)
    o_ref[...] = jnp.dot(z, w3[...], preferred_element_type=jnp.float32) + b3[...]


BLK = 2000
NBLK = N // BLK


def _tc_layer(h, agg2, w1, b1, w2, b2, w3, b3):
    wspec = pl.BlockSpec((F, F), lambda i: (0, 0))
    bspec = pl.BlockSpec((1, F), lambda i: (0, 0))
    return pl.pallas_call(
        _layer_body,
        grid=(NBLK,),
        in_specs=[
            pl.BlockSpec((BLK, F), lambda i: (i, 0)),
            pl.BlockSpec((NC, BLK, F), lambda i: (0, i, 0)),
            wspec, bspec, wspec, bspec, wspec, bspec,
        ],
        out_specs=pl.BlockSpec((BLK, F), lambda i: (i, 0)),
        out_shape=jax.ShapeDtypeStruct((N, F), jnp.float32),
    )(h, agg2, w1, b1, w2, b2, w3, b3)


def _pool_body(h_ref, seg_ref, d1w, d1b, d2w, d2b, o_ref, sums, cnts):
    i = pl.program_id(0)

    @pl.when(i == 0)
    def _():
        sums[...] = jnp.zeros((G, F), jnp.float32)
        cnts[...] = jnp.zeros((G, F), jnp.float32)

    onehot = (seg_ref[...] == lax.broadcasted_iota(jnp.int32, (1, G), 1)
              ).astype(jnp.float32)
    cdims = (((0,), (0,)), ((), ()))
    sums[...] += lax.dot_general(onehot, h_ref[...], cdims,
                                 preferred_element_type=jnp.float32)
    cnts[...] += lax.dot_general(onehot, jnp.ones((BLK, F), jnp.float32), cdims,
                                 preferred_element_type=jnp.float32)

    @pl.when(i == NBLK - 1)
    def _():
        pooled = sums[...] / jnp.maximum(cnts[...], 1.0)
        o1 = jnp.maximum(
            jnp.dot(pooled, d1w[...], preferred_element_type=jnp.float32) + d1b[...], 0.0)
        logits = jnp.dot(o1, d2w[...], preferred_element_type=jnp.float32) + d2b[...]
        mask = lax.broadcasted_iota(jnp.int32, (G, F), 1) < N_OUT
        logits = jnp.where(mask, logits, -1e30)
        m = jnp.max(logits, axis=1, keepdims=True)
        e = jnp.exp(logits - m)
        o_ref[...] = e / jnp.sum(e, axis=1, keepdims=True)


def _tc_pool_head(h, segf, d1w, d1b, d2wp, d2bp):
    wspec = pl.BlockSpec((F, F), lambda i: (0, 0))
    bspec = pl.BlockSpec((1, F), lambda i: (0, 0))
    return pl.pallas_call(
        _pool_body,
        grid=(NBLK,),
        in_specs=[
            pl.BlockSpec((BLK, F), lambda i: (i, 0)),
            pl.BlockSpec((BLK, 1), lambda i: (i, 0)),
            wspec, bspec, wspec, bspec,
        ],
        out_specs=pl.BlockSpec((G, F), lambda i: (0, 0)),
        out_shape=jax.ShapeDtypeStruct((G, F), jnp.float32),
        scratch_shapes=[
            pltpu.VMEM((G, F), jnp.float32),
            pltpu.VMEM((G, F), jnp.float32),
        ],
    )(h, segf, d1w, d1b, d2wp, d2bp)


def kernel(x, convW1, convb1, convW2, convb2, convW3, convb3,
           d1W, d1b, d2W, d2b, edge_index, seg):
    src = edge_index[0]
    dst = edge_index[1]
    ar = np.arange(EPAD - E, dtype=np.int32)
    pad_src = jnp.asarray((ar % N).reshape(-1, IDXW))
    pad_dst = jnp.asarray((N + ar % (PADN - N)).reshape(-1, IDXW))
    srcp = jnp.concatenate([src.reshape(E // IDXW, IDXW), pad_src])
    dstp = jnp.concatenate([dst.reshape(E // IDXW, IDXW), pad_dst])
    zrows = jnp.zeros((RPT, F), jnp.float32)

    h = x
    for l in range(3):
        aggp = _sc_segment_sum(h, srcp, dstp, zrows)
        agg2 = aggp.reshape(NC, PADN, F)
        h = _tc_layer(h, agg2,
                      convW1[l], convb1[l].reshape(1, F),
                      convW2[l], convb2[l].reshape(1, F),
                      convW3[l], convb3[l].reshape(1, F))

    segf = seg.reshape(N, 1)
    d2wp = jnp.pad(d2W, ((0, 0), (0, F - N_OUT)))
    d2bp = jnp.pad(d2b, (0, F - N_OUT)).reshape(1, F)
    out = _tc_pool_head(h, segf, d1W, d1b.reshape(1, F), d2wp, d2bp)
    return out[:, :N_OUT]


# speedup vs baseline: 1.1428x; 1.0447x over previous
import jax
import jax.numpy as jnp
from jax import lax
from jax.experimental import pallas as pl
from jax.experimental.pallas import tpu as pltpu
from jax.experimental.pallas import tpu_sc as plsc

N = 10000
E = 320000
F = 128
G = 64
N_OUT = 10

NC = 2
NS = 16
NW = NC * NS

IDXW = 128
EROWS = E // IDXW
RPW = 80
TAIL = EROWS - (NW - 1) * RPW
PADN = 10240
RPT = PADN // NS
CH = 40


def _sc_body(h_hbm, eidx_hbm, z_hbm, out_hbm, acc, idx_s, idx_d,
             rows0, rows1, semg0, semg1):
    cid = lax.axis_index("c")
    sid = lax.axis_index("s")
    wid = sid * NC + cid

    with jax.named_scope("ph_zero"):
        pltpu.sync_copy(z_hbm, acc.at[pl.ds(sid * RPT, RPT)])
    plsc.subcore_barrier()

    def run_span(hbase, nrows):
        pltpu.sync_copy(eidx_hbm.at[0, pl.ds(hbase, nrows)],
                        idx_s.at[pl.ds(0, nrows)])
        pltpu.sync_copy(eidx_hbm.at[1, pl.ds(hbase, nrows)],
                        idx_d.at[pl.ds(0, nrows)])
        pltpu.async_copy(h_hbm.at[idx_s.at[0]], rows0, semg0)
        pltpu.async_copy(h_hbm.at[idx_s.at[1]], rows1, semg1)

        def step(i, c):
            j = 2 * i
            for off, rows, sem in ((0, rows0, semg0), (1, rows1, semg1)):
                pltpu.make_async_copy(h_hbm.at[idx_s.at[j + off]], rows, sem).wait()
                pltpu.sync_copy(rows, acc.at[idx_d.at[j + off]], add=True)

                @pl.when(j + off + 2 < nrows)
                def _():
                    pltpu.async_copy(h_hbm.at[idx_s.at[j + off + 2]], rows, sem)
            return c

        lax.fori_loop(0, nrows // 2, step, 0)

    with jax.named_scope("ph_edges"):
        @pl.when(wid < NW - 1)
        def _():
            def chunk(ci, carry):
                run_span(wid * RPW + ci * CH, CH)
                return carry

            lax.fori_loop(0, RPW // CH, chunk, 0)

        @pl.when(wid == NW - 1)
        def _():
            run_span((NW - 1) * RPW, TAIL)

    with jax.named_scope("ph_bar"):
        plsc.subcore_barrier()

    with jax.named_scope("ph_writeout"):
        pltpu.sync_copy(acc.at[pl.ds(sid * RPT, RPT)],
                        out_hbm.at[pl.ds(cid * PADN + sid * RPT, RPT)])


@jax.jit
def _sc_segment_sum(h, eidx3, zrows):
    mesh = plsc.VectorSubcoreMesh(core_axis_name="c", subcore_axis_name="s")
    k = pl.kernel(
        _sc_body,
        out_type=jax.ShapeDtypeStruct((NC * PADN, F), jnp.float32),
        mesh=mesh,
        scratch_types=[
            pltpu.VMEM_SHARED((PADN, F), jnp.float32),
            pltpu.VMEM((CH, IDXW), jnp.int32),
            pltpu.VMEM((CH, IDXW), jnp.int32),
        ] + [pltpu.VMEM((IDXW, F), jnp.float32)] * 2
          + [pltpu.SemaphoreType.DMA] * 2,
    )
    return k(h, eidx3, zrows)


BLK = 2000
NBLK = N // BLK


def _layer_body(h_ref, agg_ref, w1, b1, w2, b2, w3, b3, o_ref):
    z = h_ref[...] + agg_ref[0] + agg_ref[1]
    z = jnp.maximum(jnp.dot(z, w1[...], preferred_element_type=jnp.float32) + b1[...], 0.0)
    z = jnp.maximum(jnp.dot(z, w2[...], preferred_element_type=jnp.float32) + b2[...], 0.0)
    o_ref[...] = jnp.dot(z, w3[...], preferred_element_type=jnp.float32) + b3[...]


def _tc_layer(h, agg2, w1, b1, w2, b2, w3, b3):
    wspec = pl.BlockSpec((F, F), lambda i: (0, 0))
    bspec = pl.BlockSpec((1, F), lambda i: (0, 0))
    return pl.pallas_call(
        _layer_body,
        grid=(NBLK,),
        in_specs=[
            pl.BlockSpec((BLK, F), lambda i: (i, 0)),
            pl.BlockSpec((NC, BLK, F), lambda i: (0, i, 0)),
            wspec, bspec, wspec, bspec, wspec, bspec,
        ],
        out_specs=pl.BlockSpec((BLK, F), lambda i: (i, 0)),
        out_shape=jax.ShapeDtypeStruct((N, F), jnp.float32),
    )(h, agg2, w1, b1, w2, b2, w3, b3)


def _layer3_body(h_ref, agg_ref, seg_ref, w1, b1, w2, b2, w3, b3,
                 d1w, d1b, d2w, d2b, o_ref, sums, cnts):
    i = pl.program_id(0)

    @pl.when(i == 0)
    def _():
        sums[...] = jnp.zeros((G, F), jnp.float32)
        cnts[...] = jnp.zeros((G, F), jnp.float32)

    z = h_ref[...] + agg_ref[0] + agg_ref[1]
    z = jnp.maximum(jnp.dot(z, w1[...], preferred_element_type=jnp.float32) + b1[...], 0.0)
    z = jnp.maximum(jnp.dot(z, w2[...], preferred_element_type=jnp.float32) + b2[...], 0.0)
    hb = jnp.dot(z, w3[...], preferred_element_type=jnp.float32) + b3[...]

    onehot = (seg_ref[...] == lax.broadcasted_iota(jnp.int32, (1, G), 1)
              ).astype(jnp.float32)
    cdims = (((0,), (0,)), ((), ()))
    sums[...] += lax.dot_general(onehot, hb, cdims,
                                 preferred_element_type=jnp.float32)
    cnts[...] += lax.dot_general(onehot, jnp.ones((BLK, F), jnp.float32), cdims,
                                 preferred_element_type=jnp.float32)

    @pl.when(i == NBLK - 1)
    def _():
        pooled = sums[...] / jnp.maximum(cnts[...], 1.0)
        o1 = jnp.maximum(
            jnp.dot(pooled, d1w[...], preferred_element_type=jnp.float32) + d1b[...], 0.0)
        logits = jnp.dot(o1, d2w[...], preferred_element_type=jnp.float32) + d2b[...]
        mask = lax.broadcasted_iota(jnp.int32, (G, F), 1) < N_OUT
        logits = jnp.where(mask, logits, -1e30)
        m = jnp.max(logits, axis=1, keepdims=True)
        e = jnp.exp(logits - m)
        o_ref[...] = e / jnp.sum(e, axis=1, keepdims=True)


def _tc_layer3_pool_head(h, agg2, seg2, w1, b1, w2, b2, w3, b3, d1w, d1b, d2wp, d2bp):
    wspec = pl.BlockSpec((F, F), lambda i: (0, 0))
    bspec = pl.BlockSpec((1, F), lambda i: (0, 0))
    return pl.pallas_call(
        _layer3_body,
        grid=(NBLK,),
        in_specs=[
            pl.BlockSpec((BLK, F), lambda i: (i, 0)),
            pl.BlockSpec((NC, BLK, F), lambda i: (0, i, 0)),
            pl.BlockSpec((BLK, 1), lambda i: (i, 0)),
            wspec, bspec, wspec, bspec, wspec, bspec,
            wspec, bspec, wspec, bspec,
        ],
        out_specs=pl.BlockSpec((G, F), lambda i: (0, 0)),
        out_shape=jax.ShapeDtypeStruct((G, F), jnp.float32),
        scratch_shapes=[
            pltpu.VMEM((G, F), jnp.float32),
            pltpu.VMEM((G, F), jnp.float32),
        ],
    )(h, agg2, seg2, w1, b1, w2, b2, w3, b3, d1w, d1b, d2wp, d2bp)


def kernel(x, convW1, convb1, convW2, convb2, convW3, convb3,
           d1W, d1b, d2W, d2b, edge_index, seg):
    eidx3 = edge_index.reshape(2, EROWS, IDXW)
    zrows = jnp.zeros((RPT, F), jnp.float32)

    h = x
    for l in range(2):
        aggp = _sc_segment_sum(h, eidx3, zrows)
        agg2 = aggp.reshape(NC, PADN, F)
        h = _tc_layer(h, agg2,
                      convW1[l], convb1[l].reshape(1, F),
                      convW2[l], convb2[l].reshape(1, F),
                      convW3[l], convb3[l].reshape(1, F))

    aggp = _sc_segment_sum(h, eidx3, zrows)
    agg2 = aggp.reshape(NC, PADN, F)
    seg2 = seg.reshape(N, 1)
    d2wp = jnp.pad(d2W, ((0, 0), (0, F - N_OUT)))
    d2bp = jnp.pad(d2b, (0, F - N_OUT)).reshape(1, F)
    out = _tc_layer3_pool_head(
        h, agg2, seg2,
        convW1[2], convb1[2].reshape(1, F),
        convW2[2], convb2[2].reshape(1, F),
        convW3[2], convb3[2].reshape(1, F),
        d1W, d1b.reshape(1, F), d2wp, d2bp)
    return out[:, :N_OUT]


# speedup vs baseline: 1.1678x; 1.0218x over previous
import jax
import jax.numpy as jnp
from jax import lax
from jax.experimental import pallas as pl
from jax.experimental.pallas import tpu as pltpu
from jax.experimental.pallas import tpu_sc as plsc

N = 10000
E = 320000
F = 128
G = 64
N_OUT = 10

NC = 2
NS = 16
NW = NC * NS

IDXW = 128
EROWS = E // IDXW
RPW = 80
TAIL = EROWS - (NW - 1) * RPW
PADN = 10240
RPT = PADN // NS
CH = 40


def _sc_body(h_hbm, eidx_hbm, z_hbm, out_hbm, acc, idx_s, idx_d,
             rows0, rows1, semg0, semg1):
    cid = lax.axis_index("c")
    sid = lax.axis_index("s")
    wid = sid * NC + cid

    with jax.named_scope("ph_zero"):
        @pl.when(cid == 0)
        def _():
            @pl.when(sid < NS - 1)
            def _():
                pltpu.sync_copy(h_hbm.at[pl.ds(sid * RPT, RPT)],
                                acc.at[pl.ds(sid * RPT, RPT)])

            @pl.when(sid == NS - 1)
            def _():
                pltpu.sync_copy(h_hbm.at[pl.ds((NS - 1) * RPT, N - (NS - 1) * RPT)],
                                acc.at[pl.ds((NS - 1) * RPT, N - (NS - 1) * RPT)])
                pltpu.sync_copy(z_hbm.at[pl.ds(0, PADN - N)],
                                acc.at[pl.ds(N, PADN - N)])

        @pl.when(cid == 1)
        def _():
            pltpu.sync_copy(z_hbm, acc.at[pl.ds(sid * RPT, RPT)])
    plsc.subcore_barrier()

    def run_span(hbase, nrows):
        pltpu.sync_copy(eidx_hbm.at[0, pl.ds(hbase, nrows)],
                        idx_s.at[pl.ds(0, nrows)])
        pltpu.sync_copy(eidx_hbm.at[1, pl.ds(hbase, nrows)],
                        idx_d.at[pl.ds(0, nrows)])
        pltpu.async_copy(h_hbm.at[idx_s.at[0]], rows0, semg0)
        pltpu.async_copy(h_hbm.at[idx_s.at[1]], rows1, semg1)

        def step(i, c):
            j = 2 * i
            for off, rows, sem in ((0, rows0, semg0), (1, rows1, semg1)):
                pltpu.make_async_copy(h_hbm.at[idx_s.at[j + off]], rows, sem).wait()
                pltpu.sync_copy(rows, acc.at[idx_d.at[j + off]], add=True)

                @pl.when(j + off + 2 < nrows)
                def _():
                    pltpu.async_copy(h_hbm.at[idx_s.at[j + off + 2]], rows, sem)
            return c

        lax.fori_loop(0, nrows // 2, step, 0)

    with jax.named_scope("ph_edges"):
        @pl.when(wid < NW - 1)
        def _():
            def chunk(ci, carry):
                run_span(wid * RPW + ci * CH, CH)
                return carry

            lax.fori_loop(0, RPW // CH, chunk, 0)

        @pl.when(wid == NW - 1)
        def _():
            run_span((NW - 1) * RPW, TAIL)

    with jax.named_scope("ph_bar"):
        plsc.subcore_barrier()

    with jax.named_scope("ph_writeout"):
        pltpu.sync_copy(acc.at[pl.ds(sid * RPT, RPT)],
                        out_hbm.at[pl.ds(cid * PADN + sid * RPT, RPT)])


@jax.jit
def _sc_segment_sum(h, eidx3, zrows):
    mesh = plsc.VectorSubcoreMesh(core_axis_name="c", subcore_axis_name="s")
    k = pl.kernel(
        _sc_body,
        out_type=jax.ShapeDtypeStruct((NC * PADN, F), jnp.float32),
        mesh=mesh,
        scratch_types=[
            pltpu.VMEM_SHARED((PADN, F), jnp.float32),
            pltpu.VMEM((CH, IDXW), jnp.int32),
            pltpu.VMEM((CH, IDXW), jnp.int32),
        ] + [pltpu.VMEM((IDXW, F), jnp.float32)] * 2
          + [pltpu.SemaphoreType.DMA] * 2,
    )
    return k(h, eidx3, zrows)


BLK = 2000
NBLK = N // BLK


def _layer_body(agg_ref, w1, b1, w2, b2, w3, b3, o_ref):
    z = agg_ref[0] + agg_ref[1]
    z = jnp.maximum(jnp.dot(z, w1[...], preferred_element_type=jnp.float32) + b1[...], 0.0)
    z = jnp.maximum(jnp.dot(z, w2[...], preferred_element_type=jnp.float32) + b2[...], 0.0)
    o_ref[...] = jnp.dot(z, w3[...], preferred_element_type=jnp.float32) + b3[...]


def _tc_layer(agg2, w1, b1, w2, b2, w3, b3):
    wspec = pl.BlockSpec((F, F), lambda i: (0, 0))
    bspec = pl.BlockSpec((1, F), lambda i: (0, 0))
    return pl.pallas_call(
        _layer_body,
        grid=(NBLK,),
        in_specs=[
            pl.BlockSpec((NC, BLK, F), lambda i: (0, i, 0)),
            wspec, bspec, wspec, bspec, wspec, bspec,
        ],
        out_specs=pl.BlockSpec((BLK, F), lambda i: (i, 0)),
        out_shape=jax.ShapeDtypeStruct((N, F), jnp.float32),
    )(agg2, w1, b1, w2, b2, w3, b3)


def _layer3_body(agg_ref, seg_ref, w1, b1, w2, b2, w3, b3,
                 d1w, d1b, d2w, d2b, o_ref, sums, cnts):
    i = pl.program_id(0)

    @pl.when(i == 0)
    def _():
        sums[...] = jnp.zeros((G, F), jnp.float32)
        cnts[...] = jnp.zeros((G, F), jnp.float32)

    z = agg_ref[0] + agg_ref[1]
    z = jnp.maximum(jnp.dot(z, w1[...], preferred_element_type=jnp.float32) + b1[...], 0.0)
    z = jnp.maximum(jnp.dot(z, w2[...], preferred_element_type=jnp.float32) + b2[...], 0.0)
    hb = jnp.dot(z, w3[...], preferred_element_type=jnp.float32) + b3[...]

    onehot = (seg_ref[...] == lax.broadcasted_iota(jnp.int32, (1, G), 1)
              ).astype(jnp.float32)
    cdims = (((0,), (0,)), ((), ()))
    sums[...] += lax.dot_general(onehot, hb, cdims,
                                 preferred_element_type=jnp.float32)
    cnts[...] += lax.dot_general(onehot, jnp.ones((BLK, F), jnp.float32), cdims,
                                 preferred_element_type=jnp.float32)

    @pl.when(i == NBLK - 1)
    def _():
        pooled = sums[...] / jnp.maximum(cnts[...], 1.0)
        o1 = jnp.maximum(
            jnp.dot(pooled, d1w[...], preferred_element_type=jnp.float32) + d1b[...], 0.0)
        logits = jnp.dot(o1, d2w[...], preferred_element_type=jnp.float32) + d2b[...]
        mask = lax.broadcasted_iota(jnp.int32, (G, F), 1) < N_OUT
        logits = jnp.where(mask, logits, -1e30)
        m = jnp.max(logits, axis=1, keepdims=True)
        e = jnp.exp(logits - m)
        o_ref[...] = e / jnp.sum(e, axis=1, keepdims=True)


def _tc_layer3_pool_head(agg2, seg2, w1, b1, w2, b2, w3, b3, d1w, d1b, d2wp, d2bp):
    wspec = pl.BlockSpec((F, F), lambda i: (0, 0))
    bspec = pl.BlockSpec((1, F), lambda i: (0, 0))
    return pl.pallas_call(
        _layer3_body,
        grid=(NBLK,),
        in_specs=[
            pl.BlockSpec((NC, BLK, F), lambda i: (0, i, 0)),
            pl.BlockSpec((BLK, 1), lambda i: (i, 0)),
            wspec, bspec, wspec, bspec, wspec, bspec,
            wspec, bspec, wspec, bspec,
        ],
        out_specs=pl.BlockSpec((G, F), lambda i: (0, 0)),
        out_shape=jax.ShapeDtypeStruct((G, F), jnp.float32),
        scratch_shapes=[
            pltpu.VMEM((G, F), jnp.float32),
            pltpu.VMEM((G, F), jnp.float32),
        ],
    )(agg2, seg2, w1, b1, w2, b2, w3, b3, d1w, d1b, d2wp, d2bp)


def kernel(x, convW1, convb1, convW2, convb2, convW3, convb3,
           d1W, d1b, d2W, d2b, edge_index, seg):
    eidx3 = edge_index.reshape(2, EROWS, IDXW)
    zrows = jnp.zeros((RPT, F), jnp.float32)

    h = x
    for l in range(2):
        aggp = _sc_segment_sum(h, eidx3, zrows)
        agg2 = aggp.reshape(NC, PADN, F)
        h = _tc_layer(agg2,
                      convW1[l], convb1[l].reshape(1, F),
                      convW2[l], convb2[l].reshape(1, F),
                      convW3[l], convb3[l].reshape(1, F))

    aggp = _sc_segment_sum(h, eidx3, zrows)
    agg2 = aggp.reshape(NC, PADN, F)
    seg2 = seg.reshape(N, 1)
    d2wp = jnp.pad(d2W, ((0, 0), (0, F - N_OUT)))
    d2bp = jnp.pad(d2b, (0, F - N_OUT)).reshape(1, F)
    out = _tc_layer3_pool_head(
        agg2, seg2,
        convW1[2], convb1[2].reshape(1, F),
        convW2[2], convb2[2].reshape(1, F),
        convW3[2], convb3[2].reshape(1, F),
        d1W, d1b.reshape(1, F), d2wp, d2bp)
    return out[:, :N_OUT]
